# baseline (speedup 1.0000x reference)
import functools

import jax
import jax.numpy as jnp
from jax import lax
from jax.experimental import pallas as pl
from jax.experimental.pallas import tpu as pltpu
from jax.experimental.pallas import tpu_sc as plsc

N = 10000
E = 320000
CIN = 128
COUT = 256
EPS = 1e-5

NC = 2
NS = 16
NW = NC * NS
EPT = E // NW
K = 128
NB = (EPT + K - 1) // K
EPAD = NB * K
HP = 10240
RPT = HP // NS
DUMMY = N + 64
NBA = 80
EPADA = NBA * K
WIN = 40
NWIN = NBA // WIN

BLK = 1000
NBLK = N // BLK

_MESH = plsc.VectorSubcoreMesh(core_axis_name="c", subcore_axis_name="s")


@functools.partial(
    pl.kernel,
    out_type=jax.ShapeDtypeStruct((NC, HP, CIN), jnp.float32),
    mesh=_MESH,
    scratch_types=[
        pltpu.VMEM((NBA, K), jnp.int32),
        pltpu.VMEM((K, CIN), jnp.float32),
        pltpu.VMEM_SHARED((HP, CIN), jnp.float32),
    ],
)
def _sc_hist(dst_hbm, ones_hbm, zeros_hbm, out_hbm, dst_v, ones_v, hist_sh):
    cid = lax.axis_index("c")
    sid = lax.axis_index("s")
    wid = cid * NS + sid

    pltpu.sync_copy(zeros_hbm, hist_sh.at[pl.ds(sid * RPT, RPT)])
    pltpu.sync_copy(dst_hbm.at[wid], dst_v)
    pltpu.sync_copy(ones_hbm, ones_v)
    plsc.subcore_barrier()

    def body(j, carry):
        pltpu.sync_copy(ones_v, hist_sh.at[dst_v.at[j]], add=True)
        return carry

    lax.fori_loop(0, NBA, body, 0)
    plsc.subcore_barrier()
    pltpu.sync_copy(
        hist_sh.at[pl.ds(sid * RPT, RPT)],
        out_hbm.at[cid, pl.ds(sid * RPT, RPT)],
    )


@functools.partial(
    pl.kernel,
    out_type=jax.ShapeDtypeStruct((NC, HP, CIN), jnp.float32),
    mesh=_MESH,
    scratch_types=[
        pltpu.VMEM((NBA, K), jnp.int32),
        pltpu.VMEM((NBA, K), jnp.int32),
        pltpu.VMEM((K, CIN), jnp.float32),
        pltpu.VMEM_SHARED((HP, CIN), jnp.float32),
    ],
)
def _sc_agg(y_hbm, src_hbm, dst_hbm, zeros_hbm, out_hbm, src_v, dst_v,
            rows_v, agg_sh):
    cid = lax.axis_index("c")
    sid = lax.axis_index("s")
    wid = cid * NS + sid

    pltpu.sync_copy(zeros_hbm, agg_sh.at[pl.ds(sid * RPT, RPT)])
    pltpu.sync_copy(src_hbm.at[wid], src_v)
    pltpu.sync_copy(dst_hbm.at[wid], dst_v)
    plsc.subcore_barrier()

    def body(j, carry):
        pltpu.sync_copy(y_hbm.at[src_v.at[j]], rows_v)
        pltpu.sync_copy(rows_v, agg_sh.at[dst_v.at[j]], add=True)
        return carry

    lax.fori_loop(0, NBA, body, 0)
    plsc.subcore_barrier()
    pltpu.sync_copy(
        agg_sh.at[pl.ds(sid * RPT, RPT)],
        out_hbm.at[cid, pl.ds(sid * RPT, RPT)],
    )


def _tc_prep_body(hist_ref, x_ref, y_ref, dis_ref):
    h = hist_ref[...]
    deg = h[0, :, 0:1] + h[1, :, 0:1] + 1.0
    dis = lax.rsqrt(deg)
    y_ref[...] = x_ref[...] * dis
    dis_ref[...] = dis


def _tc_prep(hist2, x):
    return pl.pallas_call(
        _tc_prep_body,
        grid=(NBLK,),
        in_specs=[
            pl.BlockSpec((NC, BLK, CIN), lambda i: (0, i, 0)),
            pl.BlockSpec((BLK, CIN), lambda i: (i, 0)),
        ],
        out_specs=[
            pl.BlockSpec((BLK, CIN), lambda i: (i, 0)),
            pl.BlockSpec((BLK, 1), lambda i: (i, 0)),
        ],
        out_shape=[
            jax.ShapeDtypeStruct((N, CIN), jnp.float32),
            jax.ShapeDtypeStruct((N, 1), jnp.float32),
        ],
    )(hist2, x)


def _tc_out_body(agg_ref, y_ref, dis_ref, x_ref, w_ref, p_ref,
                 g_ref, be_ref, pb_ref, o_ref, st_ref):
    p = pl.program_id(0)
    i = pl.program_id(1)
    a = (agg_ref[0] + agg_ref[1] + y_ref[...]) * dis_ref[...]
    hmat = jnp.dot(a, w_ref[...], preferred_element_type=jnp.float32)

    @pl.when(jnp.logical_and(p == 0, i == 0))
    def _():
        st_ref[...] = jnp.zeros_like(st_ref)

    @pl.when(p == 0)
    def _():
        s1 = jnp.sum(hmat, axis=0, keepdims=True)
        s2 = jnp.sum(hmat * hmat, axis=0, keepdims=True)
        st_ref[...] += jnp.concatenate([s1, s2], axis=0)

    @pl.when(p == 1)
    def _():
        st = st_ref[...]
        mean = st[0:1] / N
        var = st[1:2] / N - mean * mean
        scale = g_ref[...] * lax.rsqrt(var + EPS)
        shift = be_ref[...] - mean * scale + pb_ref[...]
        idp = lax.dot_general(
            x_ref[...], p_ref[...],
            dimension_numbers=(((1,), (1,)), ((), ())),
            preferred_element_type=jnp.float32,
        )
        o_ref[...] = jnp.maximum(hmat * scale + idp + shift, 0.0)


def _tc_out(agg2, y, dis, x, W, proj_W, gamma, beta, proj_b):
    return pl.pallas_call(
        _tc_out_body,
        grid=(2, NBLK),
        in_specs=[
            pl.BlockSpec((NC, BLK, CIN), lambda p, i: (0, i, 0)),
            pl.BlockSpec((BLK, CIN), lambda p, i: (i, 0)),
            pl.BlockSpec((BLK, 1), lambda p, i: (i, 0)),
            pl.BlockSpec((BLK, CIN), lambda p, i: (i, 0)),
            pl.BlockSpec((CIN, COUT), lambda p, i: (0, 0)),
            pl.BlockSpec((COUT, CIN), lambda p, i: (0, 0)),
            pl.BlockSpec((1, COUT), lambda p, i: (0, 0)),
            pl.BlockSpec((1, COUT), lambda p, i: (0, 0)),
            pl.BlockSpec((1, COUT), lambda p, i: (0, 0)),
        ],
        out_specs=pl.BlockSpec((BLK, COUT), lambda p, i: (i, 0)),
        out_shape=jax.ShapeDtypeStruct((N, COUT), jnp.float32),
        scratch_shapes=[pltpu.VMEM((2, COUT), jnp.float32)],
    )(agg2, y, dis, x, W, proj_W, gamma, beta, proj_b)


def kernel(x, edge_index, W, b, gamma, beta, proj_W, proj_b):
    del b
    src = edge_index[0].astype(jnp.int32).reshape(NW, EPT)
    dst = edge_index[1].astype(jnp.int32).reshape(NW, EPT)
    pad = EPADA - EPT
    src_p = jnp.pad(src, ((0, 0), (0, pad))).reshape(NW, NBA, K)
    dst_p = jnp.pad(dst, ((0, 0), (0, pad)),
                    constant_values=DUMMY).reshape(NW, NBA, K)

    ones_rows = jnp.ones((K, CIN), jnp.float32)
    zeros_rows = jnp.zeros((RPT, CIN), jnp.float32)

    hist2 = _sc_hist(dst_p, ones_rows, zeros_rows)
    y, dis = _tc_prep(hist2, x)
    agg2 = _sc_agg(y, src_p, dst_p, zeros_rows)
    out = _tc_out(agg2, y, dis, x, W, proj_W,
                  gamma.reshape(1, COUT), beta.reshape(1, COUT),
                  proj_b.reshape(1, COUT))
    return out


# speedup vs baseline: 1.0047x; 1.0047x over previous
import functools

import jax
import jax.numpy as jnp
from jax import lax
from jax.experimental import pallas as pl
from jax.experimental.pallas import tpu as pltpu
from jax.experimental.pallas import tpu_sc as plsc

N = 10000
E = 320000
CIN = 128
COUT = 256
EPS = 1e-5

NC = 2
NS = 16
NW = NC * NS
EPT = E // NW
K = 128
NB = (EPT + K - 1) // K
EPAD = NB * K
HP = 10240
RPT = HP // NS
DUMMY = N + 64
NBA = 80
EPADA = NBA * K
WIN = 40
NWIN = NBA // WIN

BLK = 1000
NBLK = N // BLK

_MESH = plsc.VectorSubcoreMesh(core_axis_name="c", subcore_axis_name="s")


@functools.partial(
    pl.kernel,
    out_type=jax.ShapeDtypeStruct((NC, HP, CIN), jnp.float32),
    mesh=_MESH,
    scratch_types=[
        pltpu.VMEM((NBA, K), jnp.int32),
        pltpu.VMEM((K, CIN), jnp.float32),
        pltpu.VMEM_SHARED((HP, CIN), jnp.float32),
    ],
)
def _sc_hist(dst_hbm, ones_hbm, zeros_hbm, out_hbm, dst_v, ones_v, hist_sh):
    cid = lax.axis_index("c")
    sid = lax.axis_index("s")
    wid = cid * NS + sid

    pltpu.sync_copy(zeros_hbm, hist_sh.at[pl.ds(sid * RPT, RPT)])
    pltpu.sync_copy(dst_hbm.at[wid], dst_v)
    pltpu.sync_copy(ones_hbm, ones_v)
    plsc.subcore_barrier()

    def body(j, carry):
        pltpu.sync_copy(ones_v, hist_sh.at[dst_v.at[j]], add=True)
        return carry

    lax.fori_loop(0, NBA, body, 0)
    plsc.subcore_barrier()
    pltpu.sync_copy(
        hist_sh.at[pl.ds(sid * RPT, RPT)],
        out_hbm.at[cid, pl.ds(sid * RPT, RPT)],
    )


@functools.partial(
    pl.kernel,
    out_type=jax.ShapeDtypeStruct((NC, HP, CIN), jnp.float32),
    mesh=_MESH,
    scratch_types=[
        pltpu.VMEM((NBA, K), jnp.int32),
        pltpu.VMEM((NBA, K), jnp.int32),
        pltpu.VMEM((K, CIN), jnp.float32),
        pltpu.VMEM_SHARED((HP, CIN), jnp.float32),
    ],
)
def _sc_agg(y_hbm, src_hbm, dst_hbm, zeros_hbm, out_hbm, src_v, dst_v,
            rows_v, agg_sh):
    cid = lax.axis_index("c")
    sid = lax.axis_index("s")
    wid = cid * NS + sid

    pltpu.sync_copy(zeros_hbm, agg_sh.at[pl.ds(sid * RPT, RPT)])
    pltpu.sync_copy(src_hbm.at[wid], src_v)
    pltpu.sync_copy(dst_hbm.at[wid], dst_v)
    plsc.subcore_barrier()

    def body(j, carry):
        pltpu.sync_copy(y_hbm.at[src_v.at[j]], rows_v)
        pltpu.sync_copy(rows_v, agg_sh.at[dst_v.at[j]], add=True)
        return carry

    lax.fori_loop(0, NBA, body, 0)
    plsc.subcore_barrier()
    pltpu.sync_copy(
        agg_sh.at[pl.ds(sid * RPT, RPT)],
        out_hbm.at[cid, pl.ds(sid * RPT, RPT)],
    )


def _tc_prep_body(hist_ref, x_ref, y_ref, dis_ref):
    h = hist_ref[...]
    deg = h[0, :, 0:1] + h[1, :, 0:1] + 1.0
    dis = lax.rsqrt(deg)
    y_ref[...] = x_ref[...] * dis
    dis_ref[...] = dis


def _tc_prep(hist2, x):
    return pl.pallas_call(
        _tc_prep_body,
        grid=(NBLK,),
        in_specs=[
            pl.BlockSpec((NC, BLK, CIN), lambda i: (0, i, 0)),
            pl.BlockSpec((BLK, CIN), lambda i: (i, 0)),
        ],
        out_specs=[
            pl.BlockSpec((BLK, CIN), lambda i: (i, 0)),
            pl.BlockSpec((BLK, 1), lambda i: (i, 0)),
        ],
        out_shape=[
            jax.ShapeDtypeStruct((N, CIN), jnp.float32),
            jax.ShapeDtypeStruct((N, 1), jnp.float32),
        ],
    )(hist2, x)


def _tc_out_body(agg_ref, y_ref, dis_ref, x_ref, w_ref, p_ref,
                 g_ref, be_ref, pb_ref, o_ref, st_ref):
    p = pl.program_id(0)
    i = pl.program_id(1)
    a = (agg_ref[0] + agg_ref[1] + y_ref[...]) * dis_ref[...]
    hmat = jnp.dot(a, w_ref[...], preferred_element_type=jnp.float32)

    @pl.when(jnp.logical_and(p == 0, i == 0))
    def _():
        st_ref[...] = jnp.zeros_like(st_ref)

    @pl.when(p == 0)
    def _():
        s1 = jnp.sum(hmat, axis=0, keepdims=True)
        s2 = jnp.sum(hmat * hmat, axis=0, keepdims=True)
        st_ref[...] += jnp.concatenate([s1, s2], axis=0)

    @pl.when(p == 1)
    def _():
        st = st_ref[...]
        mean = st[0:1] / N
        var = st[1:2] / N - mean * mean
        scale = g_ref[...] * lax.rsqrt(var + EPS)
        shift = be_ref[...] - mean * scale + pb_ref[...]
        idp = lax.dot_general(
            x_ref[...], p_ref[...],
            dimension_numbers=(((1,), (1,)), ((), ())),
            preferred_element_type=jnp.float32,
        )
        o_ref[...] = jnp.maximum(hmat * scale + idp + shift, 0.0)


def _tc_out(agg2, y, dis, x, W, proj_W, gamma, beta, proj_b):
    return pl.pallas_call(
        _tc_out_body,
        grid=(2, NBLK),
        in_specs=[
            pl.BlockSpec((NC, BLK, CIN), lambda p, i: (0, i, 0)),
            pl.BlockSpec((BLK, CIN), lambda p, i: (i, 0)),
            pl.BlockSpec((BLK, 1), lambda p, i: (i, 0)),
            pl.BlockSpec((BLK, CIN), lambda p, i: (i, 0)),
            pl.BlockSpec((CIN, COUT), lambda p, i: (0, 0)),
            pl.BlockSpec((COUT, CIN), lambda p, i: (0, 0)),
            pl.BlockSpec((1, COUT), lambda p, i: (0, 0)),
            pl.BlockSpec((1, COUT), lambda p, i: (0, 0)),
            pl.BlockSpec((1, COUT), lambda p, i: (0, 0)),
        ],
        out_specs=pl.BlockSpec((BLK, COUT), lambda p, i: (i, 0)),
        out_shape=jax.ShapeDtypeStruct((N, COUT), jnp.float32),
        scratch_shapes=[pltpu.VMEM((2, COUT), jnp.float32)],
    )(agg2, y, dis, x, W, proj_W, gamma, beta, proj_b)


def kernel(x, edge_index, W, b, gamma, beta, proj_W, proj_b):
    del b
    src = edge_index[0].astype(jnp.int32).reshape(NW, EPT)
    dst = edge_index[1].astype(jnp.int32).reshape(NW, EPT)
    pad = EPADA - EPT
    pad_dst = N + (jnp.arange(pad, dtype=jnp.int32) % (HP - N))
    src_p = jnp.pad(src, ((0, 0), (0, pad))).reshape(NW, NBA, K)
    dst_p = jnp.concatenate(
        [dst, jnp.broadcast_to(pad_dst, (NW, pad))], axis=1
    ).reshape(NW, NBA, K)

    ones_rows = jnp.ones((K, CIN), jnp.float32)
    zeros_rows = jnp.zeros((RPT, CIN), jnp.float32)

    hist2 = _sc_hist(dst_p, ones_rows, zeros_rows)
    y, dis = _tc_prep(hist2, x)
    agg2 = _sc_agg(y, src_p, dst_p, zeros_rows)
    out = _tc_out(agg2, y, dis, x, W, proj_W,
                  gamma.reshape(1, COUT), beta.reshape(1, COUT),
                  proj_b.reshape(1, COUT))
    return out


# speedup vs baseline: 1.3507x; 1.3444x over previous
import functools

import jax
import jax.numpy as jnp
from jax import lax
from jax.experimental import pallas as pl
from jax.experimental.pallas import tpu as pltpu
from jax.experimental.pallas import tpu_sc as plsc

N = 10000
E = 320000
CIN = 128
COUT = 256
EPS = 1e-5

NC = 2
NS = 16
NW = NC * NS
EPT = E // NW
K = 128
NB = (EPT + K - 1) // K
EPAD = NB * K
HP = 10240
RPT = HP // NS
DUMMY = N + 64
NBA = 79
EPADA = NBA * K
WIN = 40
NWIN = NBA // WIN

BLK = 1000
NBLK = N // BLK

_MESH = plsc.VectorSubcoreMesh(core_axis_name="c", subcore_axis_name="s")


@functools.partial(
    pl.kernel,
    out_type=jax.ShapeDtypeStruct((NC, HP, CIN), jnp.float32),
    mesh=_MESH,
    scratch_types=[
        pltpu.VMEM((NBA, K), jnp.int32),
        pltpu.VMEM((K, CIN), jnp.float32),
        pltpu.VMEM_SHARED((HP, CIN), jnp.float32),
    ],
)
def _sc_hist(dst_hbm, ones_hbm, zeros_hbm, out_hbm, dst_v, ones_v, hist_sh):
    cid = lax.axis_index("c")
    sid = lax.axis_index("s")
    wid = cid * NS + sid

    pltpu.sync_copy(zeros_hbm, hist_sh.at[pl.ds(sid * RPT, RPT)])
    pltpu.sync_copy(dst_hbm.at[wid], dst_v)
    pltpu.sync_copy(ones_hbm, ones_v)
    plsc.subcore_barrier()

    def body(j, carry):
        pltpu.sync_copy(ones_v, hist_sh.at[dst_v.at[j]], add=True)
        return carry

    lax.fori_loop(0, NBA, body, 0)
    plsc.subcore_barrier()
    pltpu.sync_copy(
        hist_sh.at[pl.ds(sid * RPT, RPT)],
        out_hbm.at[cid, pl.ds(sid * RPT, RPT)],
    )


@functools.partial(
    pl.kernel,
    out_type=jax.ShapeDtypeStruct((NC, HP, CIN), jnp.float32),
    mesh=_MESH,
    scratch_types=[
        pltpu.VMEM((NBA, K), jnp.int32),
        pltpu.VMEM((NBA, K), jnp.int32),
        pltpu.VMEM((K, CIN), jnp.float32),
        pltpu.VMEM_SHARED((HP, CIN), jnp.float32),
    ],
)
def _sc_agg(y_hbm, src_hbm, dst_hbm, zeros_hbm, out_hbm, src_v, dst_v,
            rows_v, agg_sh):
    cid = lax.axis_index("c")
    sid = lax.axis_index("s")
    wid = cid * NS + sid

    pltpu.sync_copy(zeros_hbm, agg_sh.at[pl.ds(sid * RPT, RPT)])
    pltpu.sync_copy(src_hbm.at[wid], src_v)
    pltpu.sync_copy(dst_hbm.at[wid], dst_v)
    plsc.subcore_barrier()

    def body(j, carry):
        pltpu.sync_copy(y_hbm.at[src_v.at[j]], rows_v)
        pltpu.sync_copy(rows_v, agg_sh.at[dst_v.at[j]], add=True)
        return carry

    lax.fori_loop(0, NBA, body, 0)
    plsc.subcore_barrier()
    pltpu.sync_copy(
        agg_sh.at[pl.ds(sid * RPT, RPT)],
        out_hbm.at[cid, pl.ds(sid * RPT, RPT)],
    )


def _tc_prep_body(hist_ref, x_ref, y_ref, dis_ref):
    h = hist_ref[...]
    deg = h[0, :, 0:1] + h[1, :, 0:1] + 1.0
    dis = lax.rsqrt(deg)
    y_ref[...] = x_ref[...] * dis
    dis_ref[...] = dis


def _tc_prep(hist2, x):
    return pl.pallas_call(
        _tc_prep_body,
        grid=(NBLK,),
        in_specs=[
            pl.BlockSpec((NC, BLK, CIN), lambda i: (0, i, 0)),
            pl.BlockSpec((BLK, CIN), lambda i: (i, 0)),
        ],
        out_specs=[
            pl.BlockSpec((BLK, CIN), lambda i: (i, 0)),
            pl.BlockSpec((BLK, 1), lambda i: (i, 0)),
        ],
        out_shape=[
            jax.ShapeDtypeStruct((N, CIN), jnp.float32),
            jax.ShapeDtypeStruct((N, 1), jnp.float32),
        ],
    )(hist2, x)


def _tc_out_body(agg_ref, y_ref, dis_ref, x_ref, w_ref, p_ref,
                 g_ref, be_ref, pb_ref, o_ref, st_ref):
    p = pl.program_id(0)
    i = pl.program_id(1)
    a = (agg_ref[0] + agg_ref[1] + y_ref[...]) * dis_ref[...]
    hmat = jnp.dot(a, w_ref[...], preferred_element_type=jnp.float32)

    @pl.when(jnp.logical_and(p == 0, i == 0))
    def _():
        st_ref[...] = jnp.zeros_like(st_ref)

    @pl.when(p == 0)
    def _():
        s1 = jnp.sum(hmat, axis=0, keepdims=True)
        s2 = jnp.sum(hmat * hmat, axis=0, keepdims=True)
        st_ref[...] += jnp.concatenate([s1, s2], axis=0)

    @pl.when(p == 1)
    def _():
        st = st_ref[...]
        mean = st[0:1] / N
        var = st[1:2] / N - mean * mean
        scale = g_ref[...] * lax.rsqrt(var + EPS)
        shift = be_ref[...] - mean * scale + pb_ref[...]
        idp = lax.dot_general(
            x_ref[...], p_ref[...],
            dimension_numbers=(((1,), (1,)), ((), ())),
            preferred_element_type=jnp.float32,
        )
        o_ref[...] = jnp.maximum(hmat * scale + idp + shift, 0.0)


def _tc_out(agg2, y, dis, x, W, proj_W, gamma, beta, proj_b):
    return pl.pallas_call(
        _tc_out_body,
        grid=(2, NBLK),
        in_specs=[
            pl.BlockSpec((NC, BLK, CIN), lambda p, i: (0, i, 0)),
            pl.BlockSpec((BLK, CIN), lambda p, i: (i, 0)),
            pl.BlockSpec((BLK, 1), lambda p, i: (i, 0)),
            pl.BlockSpec((BLK, CIN), lambda p, i: (i, 0)),
            pl.BlockSpec((CIN, COUT), lambda p, i: (0, 0)),
            pl.BlockSpec((COUT, CIN), lambda p, i: (0, 0)),
            pl.BlockSpec((1, COUT), lambda p, i: (0, 0)),
            pl.BlockSpec((1, COUT), lambda p, i: (0, 0)),
            pl.BlockSpec((1, COUT), lambda p, i: (0, 0)),
        ],
        out_specs=pl.BlockSpec((BLK, COUT), lambda p, i: (i, 0)),
        out_shape=jax.ShapeDtypeStruct((N, COUT), jnp.float32),
        scratch_shapes=[pltpu.VMEM((2, COUT), jnp.float32)],
    )(agg2, y, dis, x, W, proj_W, gamma, beta, proj_b)


def kernel(x, edge_index, W, b, gamma, beta, proj_W, proj_b):
    del b
    src = edge_index[0].astype(jnp.int32).reshape(NW, EPT)
    dst = edge_index[1].astype(jnp.int32).reshape(NW, EPT)
    pad = EPADA - EPT
    pad_dst = N + (jnp.arange(pad, dtype=jnp.int32) % (HP - N))
    src_p = jnp.pad(src, ((0, 0), (0, pad))).reshape(NW, NBA, K)
    dst_p = jnp.concatenate(
        [dst, jnp.broadcast_to(pad_dst, (NW, pad))], axis=1
    ).reshape(NW, NBA, K)

    ones_rows = jnp.ones((K, CIN), jnp.float32)
    zeros_rows = jnp.zeros((RPT, CIN), jnp.float32)

    hist2 = _sc_hist(dst_p, ones_rows, zeros_rows)
    y, dis = _tc_prep(hist2, x)
    agg2 = _sc_agg(y, src_p, dst_p, zeros_rows)
    out = _tc_out(agg2, y, dis, x, W, proj_W,
                  gamma.reshape(1, COUT), beta.reshape(1, COUT),
                  proj_b.reshape(1, COUT))
    return out


# speedup vs baseline: 1.8906x; 1.3997x over previous
import functools

import jax
import jax.numpy as jnp
from jax import lax
from jax.experimental import pallas as pl
from jax.experimental.pallas import tpu as pltpu
from jax.experimental.pallas import tpu_sc as plsc

N = 10000
E = 320000
CIN = 128
COUT = 256
EPS = 1e-5

NC = 2
NS = 16
NW = NC * NS
EPT = E // NW
K = 128
NB = (EPT + K - 1) // K
EPAD = NB * K
HP = 10240
RPT = HP // NS
DUMMY = N + 64
NBA = 79
EPADA = NBA * K
WIN = 40
NWIN = NBA // WIN

BLK = 1000
NBLK = N // BLK

_MESH = plsc.VectorSubcoreMesh(core_axis_name="c", subcore_axis_name="s")


@functools.partial(
    pl.kernel,
    out_type=jax.ShapeDtypeStruct((NC, HP, CIN), jnp.float32),
    mesh=_MESH,
    scratch_types=[
        pltpu.VMEM((NBA, K), jnp.int32),
        pltpu.VMEM((K, CIN), jnp.float32),
        pltpu.VMEM_SHARED((HP, CIN), jnp.float32),
    ],
)
def _sc_hist(dst_hbm, ones_hbm, zeros_hbm, out_hbm, dst_v, ones_v, hist_sh):
    cid = lax.axis_index("c")
    sid = lax.axis_index("s")
    wid = cid * NS + sid

    pltpu.sync_copy(zeros_hbm, hist_sh.at[pl.ds(sid * RPT, RPT)])
    pltpu.sync_copy(dst_hbm.at[wid], dst_v)
    pltpu.sync_copy(ones_hbm, ones_v)
    plsc.subcore_barrier()

    def body(j, carry):
        pltpu.sync_copy(ones_v, hist_sh.at[dst_v.at[j]], add=True)
        return carry

    lax.fori_loop(0, NBA, body, 0)
    plsc.subcore_barrier()
    pltpu.sync_copy(
        hist_sh.at[pl.ds(sid * RPT, RPT)],
        out_hbm.at[cid, pl.ds(sid * RPT, RPT)],
    )


@functools.partial(
    pl.kernel,
    out_type=jax.ShapeDtypeStruct((NC, HP, CIN), jnp.float32),
    mesh=_MESH,
    scratch_types=[
        pltpu.VMEM((NBA, K), jnp.int32),
        pltpu.VMEM((NBA, K), jnp.int32),
        pltpu.VMEM((K, CIN), jnp.float32),
        pltpu.VMEM_SHARED((HP, CIN), jnp.float32),
    ],
)
def _sc_agg(y_hbm, src_hbm, dst_hbm, zeros_hbm, out_hbm, src_v, dst_v,
            rows_v, agg_sh):
    cid = lax.axis_index("c")
    sid = lax.axis_index("s")
    wid = cid * NS + sid

    pltpu.sync_copy(zeros_hbm, agg_sh.at[pl.ds(sid * RPT, RPT)])
    pltpu.sync_copy(src_hbm.at[wid], src_v)
    pltpu.sync_copy(dst_hbm.at[wid], dst_v)
    plsc.subcore_barrier()

    def body(j, carry):
        pltpu.sync_copy(y_hbm.at[src_v.at[j]], rows_v)
        pltpu.sync_copy(rows_v, agg_sh.at[dst_v.at[j]], add=True)
        return carry

    lax.fori_loop(0, NBA, body, 0)
    plsc.subcore_barrier()
    pltpu.sync_copy(
        agg_sh.at[pl.ds(sid * RPT, RPT)],
        out_hbm.at[cid, pl.ds(sid * RPT, RPT)],
    )


def _tc_prep_body(hist_ref, x_ref, y_ref, dis_ref):
    h = hist_ref[...]
    deg = h[0, :, 0:1] + h[1, :, 0:1] + 1.0
    dis = lax.rsqrt(deg)
    y_ref[...] = x_ref[...] * dis
    dis_ref[...] = dis


def _tc_prep(hist2, x):
    return pl.pallas_call(
        _tc_prep_body,
        grid=(NBLK,),
        in_specs=[
            pl.BlockSpec((NC, BLK, CIN), lambda i: (0, i, 0)),
            pl.BlockSpec((BLK, CIN), lambda i: (i, 0)),
        ],
        out_specs=[
            pl.BlockSpec((BLK, CIN), lambda i: (i, 0)),
            pl.BlockSpec((BLK, 1), lambda i: (i, 0)),
        ],
        out_shape=[
            jax.ShapeDtypeStruct((N, CIN), jnp.float32),
            jax.ShapeDtypeStruct((N, 1), jnp.float32),
        ],
    )(hist2, x)


def _tc_stats_body(agg_ref, y_ref, dis_ref, w_ref, o_ref):
    i = pl.program_id(0)
    a = (agg_ref[0] + agg_ref[1] + y_ref[...]) * dis_ref[...]
    hmat = jnp.dot(a, w_ref[...], preferred_element_type=jnp.float32)
    s1 = jnp.sum(hmat, axis=0, keepdims=True)
    s2 = jnp.sum(hmat * hmat, axis=0, keepdims=True)

    @pl.when(i == 0)
    def _():
        o_ref[...] = jnp.zeros_like(o_ref)

    o_ref[...] += jnp.concatenate([s1, s2], axis=0)


def _tc_stats(agg2, y, dis, W):
    return pl.pallas_call(
        _tc_stats_body,
        grid=(NBLK,),
        in_specs=[
            pl.BlockSpec((NC, BLK, CIN), lambda i: (0, i, 0)),
            pl.BlockSpec((BLK, CIN), lambda i: (i, 0)),
            pl.BlockSpec((BLK, 1), lambda i: (i, 0)),
            pl.BlockSpec((CIN, COUT), lambda i: (0, 0)),
        ],
        out_specs=pl.BlockSpec((2, COUT), lambda i: (0, 0)),
        out_shape=jax.ShapeDtypeStruct((2, COUT), jnp.float32),
    )(agg2, y, dis, W)


def _tc_final_body(st_ref, agg_ref, y_ref, dis_ref, x_ref, w_ref, p_ref,
                   g_ref, be_ref, pb_ref, o_ref):
    st = st_ref[...]
    mean = st[0:1] / N
    var = st[1:2] / N - mean * mean
    scale = g_ref[...] * lax.rsqrt(var + EPS)
    shift = be_ref[...] - mean * scale + pb_ref[...]
    a = (agg_ref[0] + agg_ref[1] + y_ref[...]) * dis_ref[...]
    hmat = jnp.dot(a, w_ref[...], preferred_element_type=jnp.float32)
    idp = lax.dot_general(
        x_ref[...], p_ref[...],
        dimension_numbers=(((1,), (1,)), ((), ())),
        preferred_element_type=jnp.float32,
    )
    o_ref[...] = jnp.maximum(hmat * scale + idp + shift, 0.0)


def _tc_final(stats, agg2, y, dis, x, W, proj_W, gamma, beta, proj_b):
    return pl.pallas_call(
        _tc_final_body,
        grid=(NBLK,),
        in_specs=[
            pl.BlockSpec((2, COUT), lambda i: (0, 0)),
            pl.BlockSpec((NC, BLK, CIN), lambda i: (0, i, 0)),
            pl.BlockSpec((BLK, CIN), lambda i: (i, 0)),
            pl.BlockSpec((BLK, 1), lambda i: (i, 0)),
            pl.BlockSpec((BLK, CIN), lambda i: (i, 0)),
            pl.BlockSpec((CIN, COUT), lambda i: (0, 0)),
            pl.BlockSpec((COUT, CIN), lambda i: (0, 0)),
            pl.BlockSpec((1, COUT), lambda i: (0, 0)),
            pl.BlockSpec((1, COUT), lambda i: (0, 0)),
            pl.BlockSpec((1, COUT), lambda i: (0, 0)),
        ],
        out_specs=pl.BlockSpec((BLK, COUT), lambda i: (i, 0)),
        out_shape=jax.ShapeDtypeStruct((N, COUT), jnp.float32),
    )(stats, agg2, y, dis, x, W, proj_W, gamma, beta, proj_b)


def kernel(x, edge_index, W, b, gamma, beta, proj_W, proj_b):
    del b
    src = edge_index[0].astype(jnp.int32).reshape(NW, EPT)
    dst = edge_index[1].astype(jnp.int32).reshape(NW, EPT)
    pad = EPADA - EPT
    pad_dst = N + (jnp.arange(pad, dtype=jnp.int32) % (HP - N))
    pad_src = jnp.arange(pad, dtype=jnp.int32) % N
    src_p = jnp.concatenate(
        [src, jnp.broadcast_to(pad_src, (NW, pad))], axis=1
    ).reshape(NW, NBA, K)
    dst_p = jnp.concatenate(
        [dst, jnp.broadcast_to(pad_dst, (NW, pad))], axis=1
    ).reshape(NW, NBA, K)

    ones_rows = jnp.ones((K, CIN), jnp.float32)
    zeros_rows = jnp.zeros((RPT, CIN), jnp.float32)

    hist2 = _sc_hist(dst_p, ones_rows, zeros_rows)
    y, dis = _tc_prep(hist2, x)
    agg2 = _sc_agg(y, src_p, dst_p, zeros_rows)
    stats = _tc_stats(agg2, y, dis, W)
    out = _tc_final(stats, agg2, y, dis, x, W, proj_W,
                    gamma.reshape(1, COUT), beta.reshape(1, COUT),
                    proj_b.reshape(1, COUT))
    return out


# speedup vs baseline: 1.8956x; 1.0026x over previous
import functools

import jax
import jax.numpy as jnp
from jax import lax
from jax.experimental import pallas as pl
from jax.experimental.pallas import tpu as pltpu
from jax.experimental.pallas import tpu_sc as plsc

N = 10000
E = 320000
CIN = 128
COUT = 256
EPS = 1e-5

NC = 2
NS = 16
NW = NC * NS
EPT = E // NW
K = 128
HP = 10240
RPT = HP // NS
NBA = (EPT + K - 1) // K
EPADA = NBA * K

BLK = 1000
NBLK = N // BLK

_MESH = plsc.VectorSubcoreMesh(core_axis_name="c", subcore_axis_name="s")


@functools.partial(
    pl.kernel,
    out_type=jax.ShapeDtypeStruct((NC, HP, CIN), jnp.float32),
    mesh=_MESH,
    scratch_types=[
        pltpu.VMEM((NBA, K), jnp.int32),
        pltpu.VMEM((K, CIN), jnp.float32),
        pltpu.VMEM_SHARED((HP, CIN), jnp.float32),
    ],
)
def _sc_hist(dst_hbm, ones_hbm, zeros_hbm, out_hbm, dst_v, ones_v, hist_sh):
    cid = lax.axis_index("c")
    sid = lax.axis_index("s")
    wid = cid * NS + sid

    pltpu.sync_copy(zeros_hbm, hist_sh.at[pl.ds(sid * RPT, RPT)])
    pltpu.sync_copy(dst_hbm.at[wid], dst_v)
    pltpu.sync_copy(ones_hbm, ones_v)
    plsc.subcore_barrier()

    def body(j, carry):
        pltpu.sync_copy(ones_v, hist_sh.at[dst_v.at[j]], add=True)
        return carry

    lax.fori_loop(0, NBA, body, 0)
    plsc.subcore_barrier()
    pltpu.sync_copy(
        hist_sh.at[pl.ds(sid * RPT, RPT)],
        out_hbm.at[cid, pl.ds(sid * RPT, RPT)],
    )


@functools.partial(
    pl.kernel,
    out_type=jax.ShapeDtypeStruct((NC, HP, CIN), jnp.float32),
    mesh=_MESH,
    scratch_types=[
        pltpu.VMEM((NBA, K), jnp.int32),
        pltpu.VMEM((NBA, K), jnp.int32),
        pltpu.VMEM((K, CIN), jnp.float32),
        pltpu.VMEM_SHARED((HP, CIN), jnp.float32),
    ],
)
def _sc_agg(y_hbm, src_hbm, dst_hbm, zeros_hbm, out_hbm, src_v, dst_v,
            rows_v, agg_sh):
    cid = lax.axis_index("c")
    sid = lax.axis_index("s")
    wid = cid * NS + sid

    pltpu.sync_copy(zeros_hbm, agg_sh.at[pl.ds(sid * RPT, RPT)])
    pltpu.sync_copy(src_hbm.at[wid], src_v)
    pltpu.sync_copy(dst_hbm.at[wid], dst_v)
    plsc.subcore_barrier()

    def body(j, carry):
        pltpu.sync_copy(y_hbm.at[src_v.at[j]], rows_v)
        pltpu.sync_copy(rows_v, agg_sh.at[dst_v.at[j]], add=True)
        return carry

    lax.fori_loop(0, NBA, body, 0)
    plsc.subcore_barrier()
    pltpu.sync_copy(
        agg_sh.at[pl.ds(sid * RPT, RPT)],
        out_hbm.at[cid, pl.ds(sid * RPT, RPT)],
    )


def _tc_prep_body(hist_ref, x_ref, y_ref, dis_ref):
    h = hist_ref[...]
    deg = h[0, :, 0:1] + h[1, :, 0:1] + 1.0
    dis = lax.rsqrt(deg)
    y_ref[...] = x_ref[...] * dis
    dis_ref[...] = dis


def _tc_prep(hist2, x):
    return pl.pallas_call(
        _tc_prep_body,
        grid=(NBLK,),
        in_specs=[
            pl.BlockSpec((NC, BLK, CIN), lambda i: (0, i, 0)),
            pl.BlockSpec((BLK, CIN), lambda i: (i, 0)),
        ],
        out_specs=[
            pl.BlockSpec((BLK, CIN), lambda i: (i, 0)),
            pl.BlockSpec((BLK, 1), lambda i: (i, 0)),
        ],
        out_shape=[
            jax.ShapeDtypeStruct((N, CIN), jnp.float32),
            jax.ShapeDtypeStruct((N, 1), jnp.float32),
        ],
    )(hist2, x)


def _tc_stats_body(agg_ref, y_ref, dis_ref, w_ref, o_ref):
    i = pl.program_id(0)
    a = (agg_ref[0] + agg_ref[1] + y_ref[...]) * dis_ref[...]
    hmat = jnp.dot(a, w_ref[...], preferred_element_type=jnp.float32)
    s1 = jnp.sum(hmat, axis=0, keepdims=True)
    s2 = jnp.sum(hmat * hmat, axis=0, keepdims=True)

    @pl.when(i == 0)
    def _():
        o_ref[...] = jnp.zeros_like(o_ref)

    o_ref[...] += jnp.concatenate([s1, s2], axis=0)


def _tc_stats(agg2, y, dis, W):
    return pl.pallas_call(
        _tc_stats_body,
        grid=(NBLK,),
        in_specs=[
            pl.BlockSpec((NC, BLK, CIN), lambda i: (0, i, 0)),
            pl.BlockSpec((BLK, CIN), lambda i: (i, 0)),
            pl.BlockSpec((BLK, 1), lambda i: (i, 0)),
            pl.BlockSpec((CIN, COUT), lambda i: (0, 0)),
        ],
        out_specs=pl.BlockSpec((2, COUT), lambda i: (0, 0)),
        out_shape=jax.ShapeDtypeStruct((2, COUT), jnp.float32),
    )(agg2, y, dis, W)


def _tc_final_body(st_ref, agg_ref, y_ref, dis_ref, x_ref, w_ref, p_ref,
                   g_ref, be_ref, pb_ref, o_ref):
    st = st_ref[...]
    mean = st[0:1] / N
    var = st[1:2] / N - mean * mean
    scale = g_ref[...] * lax.rsqrt(var + EPS)
    shift = be_ref[...] - mean * scale + pb_ref[...]
    a = (agg_ref[0] + agg_ref[1] + y_ref[...]) * dis_ref[...]
    hmat = jnp.dot(a, w_ref[...], preferred_element_type=jnp.float32)
    idp = lax.dot_general(
        x_ref[...], p_ref[...],
        dimension_numbers=(((1,), (1,)), ((), ())),
        preferred_element_type=jnp.float32,
    )
    o_ref[...] = jnp.maximum(hmat * scale + idp + shift, 0.0)


def _tc_final(stats, agg2, y, dis, x, W, proj_W, gamma, beta, proj_b):
    return pl.pallas_call(
        _tc_final_body,
        grid=(NBLK,),
        in_specs=[
            pl.BlockSpec((2, COUT), lambda i: (0, 0)),
            pl.BlockSpec((NC, BLK, CIN), lambda i: (0, i, 0)),
            pl.BlockSpec((BLK, CIN), lambda i: (i, 0)),
            pl.BlockSpec((BLK, 1), lambda i: (i, 0)),
            pl.BlockSpec((BLK, CIN), lambda i: (i, 0)),
            pl.BlockSpec((CIN, COUT), lambda i: (0, 0)),
            pl.BlockSpec((COUT, CIN), lambda i: (0, 0)),
            pl.BlockSpec((1, COUT), lambda i: (0, 0)),
            pl.BlockSpec((1, COUT), lambda i: (0, 0)),
            pl.BlockSpec((1, COUT), lambda i: (0, 0)),
        ],
        out_specs=pl.BlockSpec((BLK, COUT), lambda i: (i, 0)),
        out_shape=jax.ShapeDtypeStruct((N, COUT), jnp.float32),
    )(stats, agg2, y, dis, x, W, proj_W, gamma, beta, proj_b)


def kernel(x, edge_index, W, b, gamma, beta, proj_W, proj_b):
    del b
    src = edge_index[0].astype(jnp.int32).reshape(NW, EPT)
    dst = edge_index[1].astype(jnp.int32).reshape(NW, EPT)
    pad = EPADA - EPT
    pad_dst = N + (jnp.arange(pad, dtype=jnp.int32) % (HP - N))
    pad_src = jnp.arange(pad, dtype=jnp.int32) % N
    src_p = jnp.concatenate(
        [src, jnp.broadcast_to(pad_src, (NW, pad))], axis=1
    ).reshape(NW, NBA, K)
    dst_p = jnp.concatenate(
        [dst, jnp.broadcast_to(pad_dst, (NW, pad))], axis=1
    ).reshape(NW, NBA, K)

    ones_rows = jnp.ones((K, CIN), jnp.float32)
    zeros_rows = jnp.zeros((RPT, CIN), jnp.float32)

    hist2 = _sc_hist(dst_p, ones_rows, zeros_rows)
    y, dis = _tc_prep(hist2, x)
    agg2 = _sc_agg(y, src_p, dst_p, zeros_rows)
    stats = _tc_stats(agg2, y, dis, W)
    out = _tc_final(stats, agg2, y, dis, x, W, proj_W,
                    gamma.reshape(1, COUT), beta.reshape(1, COUT),
                    proj_b.reshape(1, COUT))
    return out
